# Initial kernel scaffold; baseline (speedup 1.0000x reference)
import jax
import jax.numpy as jnp
from jax.experimental import pallas as pl


def kernel(logits, sample_mask):
    raise NotImplementedError("write your pallas kernel here")


# speedup vs baseline: 15.8497x; 15.8497x over previous
import jax
import jax.numpy as jnp
from jax import lax
from jax.experimental import pallas as pl
from jax.experimental.pallas import tpu as pltpu

SHAPE3 = (64, 512, 512)
N = SHAPE3[0] * SHAPE3[1] * SHAPE3[2]
R, C = 8192, 2048
BR = 256
NB = R // BR
K = 12582911
S1 = 10.0
S2 = 10.0
SP = 0.25
NPASS = 7


def _mean_body(x_ref, out_ref, acc_ref):
    i = pl.program_id(0)

    @pl.when(i == 0)
    def _():
        acc_ref[0] = jnp.float32(0.0)

    acc_ref[0] += jnp.sum(jax.nn.sigmoid(S1 * x_ref[...]))

    @pl.when(i == NB - 1)
    def _():
        out_ref[0] = acc_ref[0] / N


def _stage2_body(par_ref, x_ref, s_ref, pm_ref, im_ref, hist_ref):
    i = pl.program_id(0)

    @pl.when(i == 0)
    def _():
        for j in range(16):
            hist_ref[j] = jnp.int32(0)

    r = par_ref[0]
    beta = par_ref[1]
    le = par_ref[2]
    probs = jax.nn.sigmoid(S1 * x_ref[...])
    pm = le * (probs * r) + (1.0 - le) * (1.0 - (1.0 - probs) * beta)
    pm_ref[...] = pm
    im = jax.nn.sigmoid(S2 * (pm - s_ref[...]))
    im_ref[...] = im
    key = lax.bitcast_convert_type(im, jnp.int32)
    nib = (key >> 28) & 15
    for j in range(16):
        hist_ref[j] += jnp.sum((nib == j).astype(jnp.int32))


def _decide(hist_ref, rank):
    cum = jnp.int32(0)
    b = jnp.int32(0)
    base = jnp.int32(0)
    for j in range(16):
        cnt = hist_ref[j]
        cum = cum + cnt
        t = (cum <= rank).astype(jnp.int32)
        b = b + t
        base = base + t * cnt
    return b, rank - base


def _select_body(h0_ref, im_ref, th_ref, hist_ref, st_ref):
    p = pl.program_id(0)
    i = pl.program_id(1)

    @pl.when((p == 0) & (i == 0))
    def _():
        b, nrank = _decide(h0_ref, jnp.int32(K))
        st_ref[0] = b
        st_ref[1] = nrank
        for j in range(16):
            hist_ref[j] = jnp.int32(0)

    shift = 24 - 4 * p
    prefix = st_ref[0]
    key = lax.bitcast_convert_type(im_ref[...], jnp.int32)
    match = (key >> (shift + 4)) == prefix
    nib = (key >> shift) & 15
    for j in range(16):
        hist_ref[j] += jnp.sum((match & (nib == j)).astype(jnp.int32))

    @pl.when(i == NB - 1)
    def _():
        b, nrank = _decide(hist_ref, st_ref[1])
        st_ref[0] = (st_ref[0] << 4) | b
        st_ref[1] = nrank
        for j in range(16):
            hist_ref[j] = jnp.int32(0)

        @pl.when(p == NPASS - 1)
        def _():
            th_ref[0] = lax.bitcast_convert_type(st_ref[0], jnp.float32)


def _final_body(th_ref, im_ref, out_ref):
    out_ref[...] = (im_ref[...] >= th_ref[0]).astype(jnp.float32)


def _run(x, s, interpret=False):
    blk = pl.BlockSpec((BR, C), lambda i: (i, 0))
    smem = pl.BlockSpec(memory_space=pltpu.SMEM)

    xbar = pl.pallas_call(
        _mean_body,
        grid=(NB,),
        in_specs=[blk],
        out_specs=smem,
        out_shape=jax.ShapeDtypeStruct((1,), jnp.float32),
        scratch_shapes=[pltpu.SMEM((1,), jnp.float32)],
        interpret=interpret,
    )(x)[0]

    r = SP / xbar
    beta = (1.0 - SP) / (1.0 - xbar)
    le = (r <= 1.0).astype(jnp.float32)
    params = jnp.stack([r, beta, le])

    pm, im, hist0 = pl.pallas_call(
        _stage2_body,
        grid=(NB,),
        in_specs=[smem, blk, blk],
        out_specs=[blk, blk, smem],
        out_shape=[
            jax.ShapeDtypeStruct((R, C), jnp.float32),
            jax.ShapeDtypeStruct((R, C), jnp.float32),
            jax.ShapeDtypeStruct((16,), jnp.int32),
        ],
        interpret=interpret,
    )(params, x, s)

    blk2 = pl.BlockSpec((BR, C), lambda p, i: (i, 0))
    thresh = pl.pallas_call(
        _select_body,
        grid=(NPASS, NB),
        in_specs=[smem, blk2],
        out_specs=smem,
        out_shape=jax.ShapeDtypeStruct((1,), jnp.float32),
        scratch_shapes=[pltpu.SMEM((16,), jnp.int32),
                        pltpu.SMEM((2,), jnp.int32)],
        interpret=interpret,
    )(hist0, im)

    fm = pl.pallas_call(
        _final_body,
        grid=(NB,),
        in_specs=[smem, blk],
        out_specs=blk,
        out_shape=jax.ShapeDtypeStruct((R, C), jnp.float32),
        input_output_aliases={1: 0},
        interpret=interpret,
    )(thresh, im)

    return pm, fm


def kernel(logits, sample_mask):
    x = logits.reshape(R, C)
    s = sample_mask.reshape(R, C)
    pm, fm = _run(x, s)
    return (pm.reshape(SHAPE3), fm.reshape(SHAPE3))


# speedup vs baseline: 21.9388x; 1.3842x over previous
import functools

import jax
import jax.numpy as jnp
from jax import lax
from jax.experimental import pallas as pl
from jax.experimental.pallas import tpu as pltpu
from jax.experimental.pallas import tpu_sc as plsc

SHAPE3 = (64, 512, 512)
N = SHAPE3[0] * SHAPE3[1] * SHAPE3[2]
R, C = 8192, 2048
BR = 256
NB = R // BR
K = 12582911
S1 = 10.0
S2 = 10.0
SP = 0.25

NC, NS, L = 2, 16, 16
NW = NC * NS
NPW = N // NW
CHUNK = 16384
NCH = NPW // CHUNK
UNROLL = 8
HB = 4096
HL = HB * L


def _mean_body(x_ref, out_ref, acc_ref):
    i = pl.program_id(0)

    @pl.when(i == 0)
    def _():
        acc_ref[0] = jnp.float32(0.0)

    acc_ref[0] += jnp.sum(jax.nn.sigmoid(S1 * x_ref[...]))

    @pl.when(i == NB - 1)
    def _():
        out_ref[0] = acc_ref[0] / N


def _stage2_body(par_ref, x_ref, s_ref, pm_ref, im_ref):
    r = par_ref[0]
    beta = par_ref[1]
    le = par_ref[2]
    probs = jax.nn.sigmoid(S1 * x_ref[...])
    pm = le * (probs * r) + (1.0 - le) * (1.0 - (1.0 - probs) * beta)
    pm_ref[...] = pm
    im = jax.nn.sigmoid(S2 * (pm - s_ref[...]))
    im_ref[...] = lax.bitcast_convert_type(im, jnp.int32)


def _final_body(th_ref, im_ref, out_ref):
    out_ref[...] = (im_ref[...] >= th_ref[0]).astype(jnp.float32)


def _make_sc_hist(shift, bmask, mshift, need_match):
    mesh = plsc.VectorSubcoreMesh(
        core_axis_name="c", subcore_axis_name="s",
        num_cores=NC, num_subcores=NS)

    @functools.partial(
        pl.kernel,
        out_type=jax.ShapeDtypeStruct((NW, HL), jnp.int32),
        mesh=mesh,
        compiler_params=pltpu.CompilerParams(needs_layout_passes=False),
        scratch_types=[
            pltpu.VMEM((CHUNK,), jnp.int32),
            pltpu.VMEM((CHUNK,), jnp.int32),
            pltpu.VMEM((HL,), jnp.int32),
            pltpu.VMEM((L,), jnp.int32),
            pltpu.SemaphoreType.DMA,
            pltpu.SemaphoreType.DMA,
        ],
    )
    def sc_hist(im_hbm, pref_hbm, out_hbm, buf0, buf1, hist, pref_v, sem0, sem1):
        wid = lax.axis_index("c") * NS + lax.axis_index("s")
        base = wid * NPW

        def zbody(i, _):
            hist[pl.ds(i * L, L)] = jnp.zeros((L,), jnp.int32)
            return 0

        lax.fori_loop(0, HB, zbody, 0, unroll=4)

        pltpu.sync_copy(pref_hbm, pref_v)
        pv = pref_v[...]
        lane = lax.iota(jnp.int32, L)
        ones = jnp.ones((L,), jnp.int32)

        def dma(c, buf, sem):
            return pltpu.make_async_copy(
                im_hbm.at[pl.ds(base + c * CHUNK, CHUNK)], buf, sem
            )

        def proc(buf):
            def ibody(k, _):
                for u in range(UNROLL):
                    key = buf[pl.ds((k * UNROLL + u) * L, L)]
                    nib = (key >> shift) & bmask
                    idx = lane * HB + nib
                    if need_match:
                        match = (key >> mshift) == pv
                        plsc.addupdate_scatter(hist, [idx], ones, mask=match)
                    else:
                        plsc.addupdate_scatter(hist, [idx], ones)
                return 0

            lax.fori_loop(0, CHUNK // (L * UNROLL), ibody, 0)

        dma(0, buf0, sem0).start()
        dma(1, buf1, sem1).start()

        def cbody(i, _):
            c = i * 2
            dma(c, buf0, sem0).wait()
            proc(buf0)

            @pl.when(c + 2 < NCH)
            def _():
                dma(c + 2, buf0, sem0).start()

            dma(c + 1, buf1, sem1).wait()
            proc(buf1)

            @pl.when(c + 3 < NCH)
            def _():
                dma(c + 3, buf1, sem1).start()

            return 0

        lax.fori_loop(0, NCH // 2, cbody, 0, unroll=False)

        pltpu.sync_copy(hist, out_hbm.at[wid])

    return sc_hist


def _iota2(shape, dim):
    return lax.broadcasted_iota(jnp.int32, shape, dim)


def _make_decide(bits):
    def body(st_ref, a_ref, out_ref):
        a3 = a_ref[...].reshape(NW * L, 32, 128)
        m = jnp.sum(a3, axis=0)
        rank = st_ref[1]

        rowsum = jnp.sum(m, axis=1, keepdims=True)
        io = _iota2((32, 32), 0)
        jo = _iota2((32, 32), 1)
        rs_cols = jnp.broadcast_to(rowsum, (32, 32))
        rs_rows = jnp.sum(jnp.where(io == jo, rs_cols, 0), axis=0, keepdims=True)
        rc_col = jnp.sum(
            jnp.where(jo <= io, jnp.broadcast_to(rs_rows, (32, 32)), 0),
            axis=1, keepdims=True)
        below_r = rc_col <= rank
        r_star = jnp.sum(below_r.astype(jnp.int32))
        rank2 = rank - jnp.sum(jnp.where(below_r, rowsum, 0))

        rowvec = jnp.sum(
            jnp.where(_iota2((32, 128), 0) == r_star, m, 0),
            axis=0, keepdims=True)
        io8 = _iota2((128, 128), 0)
        jo8 = _iota2((128, 128), 1)
        rv_b = jnp.broadcast_to(rowvec, (128, 128))
        rv_t = jnp.sum(jnp.where(io8 == jo8, rv_b, 0), axis=1, keepdims=True)
        cum_b = jnp.sum(
            jnp.where(io8 <= jo8, jnp.broadcast_to(rv_t, (128, 128)), 0),
            axis=0, keepdims=True)
        below_j = cum_b <= rank2
        j_star = jnp.sum(below_j.astype(jnp.int32))
        rank3 = rank2 - jnp.sum(jnp.where(below_j, rowvec, 0))

        out_ref[0] = (st_ref[0] << bits) | (r_star * 128 + j_star)
        out_ref[1] = rank3

    return body


def _decide_call(bits, st, hist, interpret=False):
    smem = pl.BlockSpec(memory_space=pltpu.SMEM)
    return pl.pallas_call(
        _make_decide(bits),
        in_specs=[smem, pl.BlockSpec((NW * L * 32, 128), lambda: (0, 0))],
        out_specs=smem,
        out_shape=jax.ShapeDtypeStruct((2,), jnp.int32),
        interpret=interpret,
    )(st, hist.reshape(NW * L * 32, 128))


_PASSES = (
    (20, HB - 1, 31, False, 12),
    (8, HB - 1, 20, True, 12),
    (0, 255, 8, True, 8),
)

@functools.lru_cache(maxsize=None)
def _sc_hist_for_pass(pi):
    sh, bm, ms, nm, _ = _PASSES[pi]
    return _make_sc_hist(sh, bm, ms, nm)


def _select_threshold(im_flat, interpret=False, hist_fn=None):
    st = jnp.array([0, K], dtype=jnp.int32)
    for pi, (sh, bm, ms, nm, bits) in enumerate(_PASSES):
        pref = jnp.full((L,), st[0], dtype=jnp.int32)
        if hist_fn is not None:
            hist = hist_fn(im_flat, pref, sh, bm, ms, nm)
        else:
            hist = _sc_hist_for_pass(pi)(im_flat, pref)
        st = _decide_call(bits, st, hist, interpret=interpret)
    return st[0:1]


def _run(x, s, interpret=False, hist_fn=None):
    blk = pl.BlockSpec((BR, C), lambda i: (i, 0))
    smem = pl.BlockSpec(memory_space=pltpu.SMEM)

    xbar = pl.pallas_call(
        _mean_body,
        grid=(NB,),
        in_specs=[blk],
        out_specs=smem,
        out_shape=jax.ShapeDtypeStruct((1,), jnp.float32),
        scratch_shapes=[pltpu.SMEM((1,), jnp.float32)],
        interpret=interpret,
    )(x)[0]

    r = SP / xbar
    beta = (1.0 - SP) / (1.0 - xbar)
    le = (r <= 1.0).astype(jnp.float32)
    params = jnp.stack([r, beta, le])

    pm, im = pl.pallas_call(
        _stage2_body,
        grid=(NB,),
        in_specs=[smem, blk, blk],
        out_specs=[blk, blk],
        out_shape=[
            jax.ShapeDtypeStruct((R, C), jnp.float32),
            jax.ShapeDtypeStruct((R, C), jnp.int32),
        ],
        interpret=interpret,
    )(params, x, s)

    thresh = _select_threshold(im.reshape(N), interpret=interpret, hist_fn=hist_fn)

    fm = pl.pallas_call(
        _final_body,
        grid=(NB,),
        in_specs=[smem, blk],
        out_specs=blk,
        out_shape=jax.ShapeDtypeStruct((R, C), jnp.float32),
        interpret=interpret,
    )(thresh, im)

    return pm, fm


def kernel(logits, sample_mask):
    x = logits.reshape(R, C)
    s = sample_mask.reshape(R, C)
    pm, fm = _run(x, s)
    return (pm.reshape(SHAPE3), fm.reshape(SHAPE3))
